# baseline (speedup 1.0000x reference)
import functools

import jax
import jax.numpy as jnp
from jax import lax
from jax.experimental import pallas as pl
from jax.experimental.pallas import tpu as pltpu
from jax.experimental.pallas import tpu_sc as plsc

_N_TABLES = 8
_VOCAB = 100000
_DIM = 64
_BATCH = 4096
_TOTAL = _N_TABLES * _BATCH

_info = plsc.get_sparse_core_info()
_NC, _NS, _L = _info.num_cores, _info.num_subcores, _info.num_lanes
_NW = _NC * _NS
_BPW = _TOTAL // _NW
_W_PER_TABLE = _BATCH // _BPW


@functools.partial(
    pl.kernel,
    out_type=jax.ShapeDtypeStruct((_TOTAL, _DIM), jnp.float32),
    mesh=plsc.VectorSubcoreMesh(core_axis_name="c", subcore_axis_name="s"),
    scratch_types=[
        pltpu.VMEM((_BPW,), jnp.int32),
        pltpu.VMEM((_BPW, _DIM), jnp.float32),
        pltpu.SemaphoreType.DMA,
    ],
    compiler_params=pltpu.CompilerParams(use_tc_tiling_on_sc=False),
)
def _gather_kernel(ids_hbm, tables_hbm, out_hbm, idx_v, rows_v, sem):
    wid = lax.axis_index("s") * _NC + lax.axis_index("c")
    base = wid * _BPW
    pltpu.sync_copy(ids_hbm.at[pl.ds(base, _BPW)], idx_v)

    offset = (wid // _W_PER_TABLE) * _VOCAB

    def _add_offset(j, carry):
        sl = pl.ds(j * _L, _L)
        idx_v[sl] = idx_v[sl] + offset
        return carry

    lax.fori_loop(0, _BPW // _L, _add_offset, 0)

    pltpu.async_copy(tables_hbm.at[idx_v], rows_v, sem).wait()
    pltpu.sync_copy(rows_v, out_hbm.at[pl.ds(base, _BPW)])


def kernel(ids, tables):
    flat_ids = ids.reshape(_TOTAL)
    flat_tables = tables.reshape(_N_TABLES * _VOCAB, _DIM)
    return _gather_kernel(flat_ids, flat_tables)


# speedup vs baseline: 4.1930x; 4.1930x over previous
import functools

import jax
import jax.numpy as jnp
from jax import lax
from jax.experimental import pallas as pl
from jax.experimental.pallas import tpu as pltpu
from jax.experimental.pallas import tpu_sc as plsc

_N_TABLES = 8
_VOCAB = 100000
_DIM = 64
_BATCH = 4096
_TOTAL = _N_TABLES * _BATCH

_info = plsc.get_sparse_core_info()
_NC, _NS, _L = _info.num_cores, _info.num_subcores, _info.num_lanes
_NW = _NC * _NS
_W_PER_TABLE = _NW // _N_TABLES
_D_PER_W = _DIM // _W_PER_TABLE

_SPLIT = 50048
_HI = _VOCAB - _SPLIT
_NCHUNK = _BATCH // _L


@functools.partial(
    pl.kernel,
    out_type=jax.ShapeDtypeStruct((_DIM, _TOTAL), jnp.float32),
    mesh=plsc.VectorSubcoreMesh(core_axis_name="c", subcore_axis_name="s"),
    scratch_types=[
        pltpu.VMEM((_BATCH,), jnp.int32),
        pltpu.VMEM((_BATCH + _L,), jnp.int32),
        pltpu.VMEM((_BATCH + _L,), jnp.int32),
        pltpu.VMEM((_SPLIT,), jnp.float32),
        pltpu.VMEM((_HI,), jnp.float32),
        pltpu.VMEM((_BATCH,), jnp.float32),
        pltpu.VMEM((_BATCH,), jnp.float32),
        pltpu.SemaphoreType.DMA,
        pltpu.SemaphoreType.DMA,
        pltpu.SemaphoreType.DMA,
        pltpu.SemaphoreType.DMA,
    ],
    compiler_params=pltpu.CompilerParams(
        use_tc_tiling_on_sc=True, needs_layout_passes=False
    ),
)
def _gather_kernel(
    ids_hbm, tables_hbm, out_hbm,
    ids_v, idx_v, pos_v, lo_v, hi_v, out0_v, out1_v,
    sem_lo, sem_hi, sem_o0, sem_o1,
):
    wid = lax.axis_index("s") * _NC + lax.axis_index("c")
    t = wid // _W_PER_TABLE
    d0 = (wid % _W_PER_TABLE) * _D_PER_W

    pltpu.sync_copy(ids_hbm.at[t], ids_v)

    def _part_lo(i, n):
        idx16 = ids_v[pl.ds(i * _L, _L)]
        pos16 = lax.iota(jnp.int32, _L) + i * _L
        m = idx16 < _SPLIT
        plsc.store_compressed(idx_v.at[pl.ds(n, _L)], idx16, mask=m)
        plsc.store_compressed(pos_v.at[pl.ds(n, _L)], pos16, mask=m)
        return n + jnp.sum(m.astype(jnp.int32))

    n_lo = lax.fori_loop(0, _NCHUNK, _part_lo, jnp.int32(0))

    def _part_hi(i, n):
        idx16 = ids_v[pl.ds(i * _L, _L)]
        pos16 = lax.iota(jnp.int32, _L) + i * _L
        m = idx16 >= _SPLIT
        plsc.store_compressed(idx_v.at[pl.ds(n, _L)], idx16 - _SPLIT, mask=m)
        plsc.store_compressed(pos_v.at[pl.ds(n, _L)], pos16, mask=m)
        return n + jnp.sum(m.astype(jnp.int32))

    lax.fori_loop(0, _NCHUNK, _part_hi, n_lo)

    n_lo_chunks = (n_lo + _L - 1) // _L
    i_hi0 = n_lo // _L

    def _stage_lo(k):
        return pltpu.async_copy(
            tables_hbm.at[t, d0 + k, pl.ds(0, _SPLIT)], lo_v, sem_lo
        )

    def _stage_hi(k):
        return pltpu.async_copy(
            tables_hbm.at[t, d0 + k, pl.ds(_SPLIT, _HI)], hi_v, sem_hi
        )

    def _gather_half(col, out_v, lo, hi, below):
        def _body(i):
            idx16 = idx_v[pl.ds(i * _L, _L)]
            pos16 = pos_v[pl.ds(i * _L, _L)]
            lane = lax.iota(jnp.int32, _L) + i * _L
            m = (lane < n_lo) if below else (lane >= n_lo)
            vals = plsc.load_gather(col, [idx16], mask=m)
            plsc.store_scatter(out_v, [pos16], vals, mask=m)

        plsc.parallel_loop(lo, hi, 1, unroll=4)(_body)

    out_bufs = (out0_v, out1_v)
    out_sems = (sem_o0, sem_o1)
    out_copies = [None, None]

    cp_lo = _stage_lo(0)
    for k in range(_D_PER_W):
        out_v = out_bufs[k % 2]
        if out_copies[k % 2] is not None:
            out_copies[k % 2].wait()
        cp_lo.wait()
        cp_hi = _stage_hi(k)
        _gather_half(lo_v, out_v, 0, n_lo_chunks, True)
        cp_hi.wait()
        if k + 1 < _D_PER_W:
            cp_lo = _stage_lo(k + 1)
        _gather_half(hi_v, out_v, i_hi0, _NCHUNK, False)
        out_copies[k % 2] = pltpu.async_copy(
            out_v, out_hbm.at[d0 + k, pl.ds(t * _BATCH, _BATCH)], out_sems[k % 2]
        )
    for c in out_copies:
        c.wait()


def kernel(ids, tables):
    tables_t = tables.transpose(0, 2, 1)
    out_t = _gather_kernel(ids, tables_t)
    return out_t.T
